# Initial kernel scaffold; baseline (speedup 1.0000x reference)
import functools
import jax
import jax.numpy as jnp
from jax import lax
from jax.experimental import pallas as pl
from jax.experimental.pallas import tpu as pltpu
from jax.experimental.pallas import tpu_sc as plsc

N = 10000
E = 160000
FH = 128
NT = 16
NB = 79
EPT = NB * 128
E_PAD = NT * EPT
ROWS_PT = N // NT
BN = 1000


def _tc_proj_body(x_ref, w_ref, as_ref, ad_ref, h_ref, sd_ref):
    h = jnp.dot(x_ref[...], w_ref[...], preferred_element_type=jnp.float32)
    h_ref[0] = h[:, :FH]
    h_ref[1] = h[:, FH:]
    s = jnp.sum(h * as_ref[...], axis=1)
    d = jnp.sum(h * ad_ref[...], axis=1)
    sd_ref[...] = jnp.concatenate(
        [s[None, :], d[None, :], jnp.zeros((6, s.shape[0]), jnp.float32)], axis=0)


def _tc_proj(x, w, a_src, a_dst):
    fin = x.shape[1]
    return pl.pallas_call(
        _tc_proj_body,
        grid=(N // BN,),
        in_specs=[
            pl.BlockSpec((BN, fin), lambda i: (i, 0)),
            pl.BlockSpec((fin, 256), lambda i: (0, 0)),
            pl.BlockSpec((1, 256), lambda i: (0, 0)),
            pl.BlockSpec((1, 256), lambda i: (0, 0)),
        ],
        out_specs=[
            pl.BlockSpec((2, BN, FH), lambda i: (0, i, 0)),
            pl.BlockSpec((8, BN), lambda i: (0, i)),
        ],
        out_shape=[
            jax.ShapeDtypeStruct((2, N, FH), jnp.float32),
            jax.ShapeDtypeStruct((8, N), jnp.float32),
        ],
    )(x, w, a_src[None, :], a_dst[None, :])


def _tc_epi_proj_body(yprev_ref, agg_ref, b_ref, w_ref, as_ref, ad_ref,
                      y_ref, h_ref, sd_ref):
    agg = jnp.concatenate([agg_ref[0], agg_ref[1]], axis=1)
    y = jnp.maximum(agg + b_ref[...], 0.0)
    if yprev_ref is not None:
        y = y + yprev_ref[...]
    y_ref[...] = y
    h = jnp.dot(y, w_ref[...], preferred_element_type=jnp.float32)
    h_ref[0] = h[:, :FH]
    h_ref[1] = h[:, FH:]
    s = jnp.sum(h * as_ref[...], axis=1)
    d = jnp.sum(h * ad_ref[...], axis=1)
    sd_ref[...] = jnp.concatenate(
        [s[None, :], d[None, :], jnp.zeros((6, s.shape[0]), jnp.float32)], axis=0)


def _tc_epi_proj(yprev, agg, b, w, a_src, a_dst):
    if yprev is not None:
        body = _tc_epi_proj_body
        args = (yprev, agg, b[None, :], w, a_src[None, :], a_dst[None, :])
        prev_specs = [pl.BlockSpec((BN, 256), lambda i: (i, 0))]
    else:
        body = functools.partial(_tc_epi_proj_body, None)
        args = (agg, b[None, :], w, a_src[None, :], a_dst[None, :])
        prev_specs = []
    return pl.pallas_call(
        body,
        grid=(N // BN,),
        in_specs=prev_specs + [
            pl.BlockSpec((2, BN, FH), lambda i: (0, i, 0)),
            pl.BlockSpec((1, 256), lambda i: (0, 0)),
            pl.BlockSpec((256, 256), lambda i: (0, 0)),
            pl.BlockSpec((1, 256), lambda i: (0, 0)),
            pl.BlockSpec((1, 256), lambda i: (0, 0)),
        ],
        out_specs=[
            pl.BlockSpec((BN, 256), lambda i: (i, 0)),
            pl.BlockSpec((2, BN, FH), lambda i: (0, i, 0)),
            pl.BlockSpec((8, BN), lambda i: (0, i)),
        ],
        out_shape=[
            jax.ShapeDtypeStruct((N, 256), jnp.float32),
            jax.ShapeDtypeStruct((2, N, FH), jnp.float32),
            jax.ShapeDtypeStruct((8, N), jnp.float32),
        ],
    )(*args)


def _tc_final_body(yprev_ref, agg_ref, b_ref, out_ref):
    agg = jnp.concatenate([agg_ref[0], agg_ref[1]], axis=1)
    out_ref[...] = yprev_ref[...] + jnp.maximum(agg + b_ref[...], 0.0)


def _tc_final(yprev, agg, b):
    return pl.pallas_call(
        _tc_final_body,
        grid=(N // BN,),
        in_specs=[
            pl.BlockSpec((BN, 256), lambda i: (i, 0)),
            pl.BlockSpec((2, BN, FH), lambda i: (0, i, 0)),
            pl.BlockSpec((1, 256), lambda i: (0, 0)),
        ],
        out_specs=pl.BlockSpec((BN, 256), lambda i: (i, 0)),
        out_shape=jax.ShapeDtypeStruct((N, 256), jnp.float32),
    )(yprev, agg, b[None, :])


def _sc_edge_body(h_hbm, s_hbm, d_hbm, src_hbm, dst_hbm, out_hbm,
                  src2d, dst2d, p2d, sloc, dloc, denloc, rowbuf,
                  denom_sh, acc_sh, sem):
    c = lax.axis_index("c")
    sid = lax.axis_index("s")
    zerov = jnp.zeros((16,), jnp.float32)

    pltpu.sync_copy(s_hbm, sloc)
    pltpu.sync_copy(d_hbm, dloc)
    pltpu.sync_copy(src_hbm.at[sid], src2d)
    pltpu.sync_copy(dst_hbm.at[sid], dst2d)

    def _zrow(i, _):
        for k in range(8):
            rowbuf[i, pl.ds(k * 16, 16)] = zerov
        return ()
    lax.fori_loop(0, 128, _zrow, ())

    def _zden(i, _):
        denloc[pl.ds(i * 16, 16)] = zerov
        return ()
    lax.fori_loop(0, N // 16, _zden, ())

    rbase = sid * ROWS_PT
    for t in range(5):
        pltpu.sync_copy(rowbuf.at[pl.ds(0, 125)],
                        acc_sh.at[pl.ds(rbase + t * 125, 125)])

    @pl.when(sid == 0)
    def _():
        pltpu.sync_copy(denloc, denom_sh)

    def _maxbody(i, carry):
        ms, md = carry
        sv = sloc[pl.ds(i * 16, 16)]
        dv = dloc[pl.ds(i * 16, 16)]
        return (jnp.maximum(ms, jnp.max(sv)), jnp.maximum(md, jnp.max(dv)))
    ms, md = lax.fori_loop(0, N // 16, _maxbody,
                           (jnp.float32(-3e38), jnp.float32(-3e38)))
    C = jnp.maximum(ms + md, 0.0)

    ebase = sid * EPT

    def _pbody(j, _):
        for k in range(8):
            sidx = src2d[j, pl.ds(k * 16, 16)]
            didx = dst2d[j, pl.ds(k * 16, 16)]
            sv = plsc.load_gather(sloc, [sidx])
            dv = plsc.load_gather(dloc, [didx])
            e = sv + dv
            e = jnp.where(e >= 0.0, e, 0.2 * e)
            p = jnp.exp(e - C)
            gid = ebase + j * 128 + k * 16 + lax.iota(jnp.int32, 16)
            p2d[j, pl.ds(k * 16, 16)] = jnp.where(gid < E, p, 0.0)
        return ()
    lax.fori_loop(0, NB, _pbody, ())

    plsc.subcore_barrier()

    def _dscat(j, _):
        pltpu.sync_copy(p2d.at[j], denom_sh.at[dst2d.at[j]], add=True)
        return ()
    lax.fori_loop(0, NB, _dscat, ())

    plsc.subcore_barrier()

    pltpu.sync_copy(denom_sh, denloc)
    off = c * N

    def _abody(j, _):
        for k in range(8):
            didx = dst2d[j, pl.ds(k * 16, 16)]
            dv = plsc.load_gather(denloc, [didx])
            pv = p2d[j, pl.ds(k * 16, 16)]
            p2d[j, pl.ds(k * 16, 16)] = pv / (dv + 1e-16)
            src2d[j, pl.ds(k * 16, 16)] = src2d[j, pl.ds(k * 16, 16)] + off
        return ()
    lax.fori_loop(0, NB, _abody, ())

    def _gbody(j, _):
        pltpu.async_copy(h_hbm.at[src2d.at[j]], rowbuf, sem).wait()

        def _mul(i, _):
            a = p2d[j, i]
            for k in range(8):
                rowbuf[i, pl.ds(k * 16, 16)] = rowbuf[i, pl.ds(k * 16, 16)] * a
            return ()
        lax.fori_loop(0, 128, _mul, ())
        pltpu.sync_copy(rowbuf, acc_sh.at[dst2d.at[j]], add=True)
        return ()
    lax.fori_loop(0, NB, _gbody, ())

    plsc.subcore_barrier()

    pltpu.sync_copy(acc_sh.at[pl.ds(rbase, ROWS_PT)],
                    out_hbm.at[pl.ds(c * N + rbase, ROWS_PT)])


_sc_edge = functools.partial(
    pl.kernel,
    mesh=plsc.VectorSubcoreMesh(core_axis_name="c", subcore_axis_name="s"),
    out_type=jax.ShapeDtypeStruct((2 * N, FH), jnp.float32),
    scratch_types=[
        pltpu.VMEM((NB, 128), jnp.int32),
        pltpu.VMEM((NB, 128), jnp.int32),
        pltpu.VMEM((NB, 128), jnp.float32),
        pltpu.VMEM((N,), jnp.float32),
        pltpu.VMEM((N,), jnp.float32),
        pltpu.VMEM((N,), jnp.float32),
        pltpu.VMEM((128, FH), jnp.float32),
        pltpu.VMEM_SHARED((N,), jnp.float32),
        pltpu.VMEM_SHARED((N, FH), jnp.float32),
        pltpu.SemaphoreType.DMA,
    ],
)(_sc_edge_body)


def _sc_layer(h_split, sd, srcp, dstp):
    h_flat = h_split.reshape(2 * N, FH)
    agg = _sc_edge(h_flat, sd[0], sd[1], srcp, dstp)
    return agg.reshape(2, N, FH)


def kernel(x, edge_index, W0, a_src0, a_dst0, b0, W1, a_src1, a_dst1, b1,
           W2, a_src2, a_dst2, b2):
    pad = jnp.zeros((E_PAD - E,), jnp.int32)
    srcp = jnp.concatenate([edge_index[0], pad]).reshape(NT, NB, 128)
    dstp = jnp.concatenate([edge_index[1], pad]).reshape(NT, NB, 128)

    h0, sd0 = _tc_proj(x, W0, a_src0, a_dst0)
    agg0 = _sc_layer(h0, sd0, srcp, dstp)
    y1, h1, sd1 = _tc_epi_proj(None, agg0, b0, W1, a_src1, a_dst1)
    agg1 = _sc_layer(h1, sd1, srcp, dstp)
    y2, h2, sd2 = _tc_epi_proj(y1, agg1, b1, W2, a_src2, a_dst2)
    agg2 = _sc_layer(h2, sd2, srcp, dstp)
    return _tc_final(y2, agg2, b2)


# speedup vs baseline: 10.2786x; 10.2786x over previous
import functools
import jax
import jax.numpy as jnp
from jax import lax
from jax.experimental import pallas as pl
from jax.experimental.pallas import tpu as pltpu
from jax.experimental.pallas import tpu_sc as plsc

N = 10000
E = 160000
FH = 128
NT = 16
NB = 79
EPT = NB * 128
E_PAD = NT * EPT
ROWS_PT = N // NT
BN = 1000


def _sd_epilogue(h, as_ref, ad_ref, sd_ref, cm_ref, acc):
    s = jnp.sum(h * as_ref[...], axis=1)
    d = jnp.sum(h * ad_ref[...], axis=1)
    sd_ref[...] = jnp.concatenate(
        [s[:, None], d[:, None], jnp.zeros((s.shape[0], 6), jnp.float32)], axis=1)
    i = pl.program_id(0)
    bs, bd = jnp.max(s), jnp.max(d)

    @pl.when(i == 0)
    def _():
        acc[0] = bs
        acc[1] = bd

    @pl.when(i > 0)
    def _():
        acc[0] = jnp.maximum(acc[0], bs)
        acc[1] = jnp.maximum(acc[1], bd)

    @pl.when(i == pl.num_programs(0) - 1)
    def _():
        cm_ref[...] = jnp.full((8, 128), jnp.maximum(acc[0] + acc[1], 0.0),
                               jnp.float32)


def _tc_proj_body(x_ref, w_ref, as_ref, ad_ref, h_ref, sd_ref, cm_ref, acc):
    h = jnp.dot(x_ref[...], w_ref[...], preferred_element_type=jnp.float32)
    h_ref[0] = h[:, :FH]
    h_ref[1] = h[:, FH:]
    _sd_epilogue(h, as_ref, ad_ref, sd_ref, cm_ref, acc)


def _tc_proj(x, w, a_src, a_dst):
    fin = x.shape[1]
    return pl.pallas_call(
        _tc_proj_body,
        grid=(N // BN,),
        in_specs=[
            pl.BlockSpec((BN, fin), lambda i: (i, 0)),
            pl.BlockSpec((fin, 256), lambda i: (0, 0)),
            pl.BlockSpec((1, 256), lambda i: (0, 0)),
            pl.BlockSpec((1, 256), lambda i: (0, 0)),
        ],
        out_specs=[
            pl.BlockSpec((2, BN, FH), lambda i: (0, i, 0)),
            pl.BlockSpec((BN, 8), lambda i: (i, 0)),
            pl.BlockSpec((8, 128), lambda i: (0, 0)),
        ],
        out_shape=[
            jax.ShapeDtypeStruct((2, N, FH), jnp.float32),
            jax.ShapeDtypeStruct((N, 8), jnp.float32),
            jax.ShapeDtypeStruct((8, 128), jnp.float32),
        ],
        scratch_shapes=[pltpu.SMEM((2,), jnp.float32)],
    )(x, w, a_src[None, :], a_dst[None, :])


def _tc_epi_proj_body(yprev_ref, agg_ref, b_ref, w_ref, as_ref, ad_ref,
                      y_ref, h_ref, sd_ref, cm_ref, acc):
    agg = jnp.concatenate([agg_ref[0], agg_ref[1]], axis=1)
    y = jnp.maximum(agg + b_ref[...], 0.0)
    if yprev_ref is not None:
        y = y + yprev_ref[...]
    y_ref[...] = y
    h = jnp.dot(y, w_ref[...], preferred_element_type=jnp.float32)
    h_ref[0] = h[:, :FH]
    h_ref[1] = h[:, FH:]
    _sd_epilogue(h, as_ref, ad_ref, sd_ref, cm_ref, acc)


def _tc_epi_proj(yprev, agg, b, w, a_src, a_dst):
    if yprev is not None:
        body = _tc_epi_proj_body
        args = (yprev, agg, b[None, :], w, a_src[None, :], a_dst[None, :])
        prev_specs = [pl.BlockSpec((BN, 256), lambda i: (i, 0))]
    else:
        body = functools.partial(_tc_epi_proj_body, None)
        args = (agg, b[None, :], w, a_src[None, :], a_dst[None, :])
        prev_specs = []
    return pl.pallas_call(
        body,
        grid=(N // BN,),
        in_specs=prev_specs + [
            pl.BlockSpec((2, BN, FH), lambda i: (0, i, 0)),
            pl.BlockSpec((1, 256), lambda i: (0, 0)),
            pl.BlockSpec((256, 256), lambda i: (0, 0)),
            pl.BlockSpec((1, 256), lambda i: (0, 0)),
            pl.BlockSpec((1, 256), lambda i: (0, 0)),
        ],
        out_specs=[
            pl.BlockSpec((BN, 256), lambda i: (i, 0)),
            pl.BlockSpec((2, BN, FH), lambda i: (0, i, 0)),
            pl.BlockSpec((BN, 8), lambda i: (i, 0)),
            pl.BlockSpec((8, 128), lambda i: (0, 0)),
        ],
        out_shape=[
            jax.ShapeDtypeStruct((N, 256), jnp.float32),
            jax.ShapeDtypeStruct((2, N, FH), jnp.float32),
            jax.ShapeDtypeStruct((N, 8), jnp.float32),
            jax.ShapeDtypeStruct((8, 128), jnp.float32),
        ],
        scratch_shapes=[pltpu.SMEM((2,), jnp.float32)],
    )(*args)


def _tc_final_body(yprev_ref, agg_ref, b_ref, out_ref):
    agg = jnp.concatenate([agg_ref[0], agg_ref[1]], axis=1)
    out_ref[...] = yprev_ref[...] + jnp.maximum(agg + b_ref[...], 0.0)


def _tc_final(yprev, agg, b):
    return pl.pallas_call(
        _tc_final_body,
        grid=(N // BN,),
        in_specs=[
            pl.BlockSpec((BN, 256), lambda i: (i, 0)),
            pl.BlockSpec((2, BN, FH), lambda i: (0, i, 0)),
            pl.BlockSpec((1, 256), lambda i: (0, 0)),
        ],
        out_specs=pl.BlockSpec((BN, 256), lambda i: (i, 0)),
        out_shape=jax.ShapeDtypeStruct((N, 256), jnp.float32),
    )(yprev, agg, b[None, :])


def _sc_edge_body(h_hbm, s_hbm, d_hbm, c_hbm, src_hbm, dst_hbm, out_hbm,
                  src2d, dst2d, p2d, svals, dvals, zbuf, rowbuf, cloc,
                  denom_sh, acc_sh, sem):
    c = lax.axis_index("c")
    sid = lax.axis_index("s")
    zerov = jnp.zeros((16,), jnp.float32)

    pltpu.sync_copy(src_hbm.at[sid], src2d)
    pltpu.sync_copy(dst_hbm.at[sid], dst2d)
    pltpu.sync_copy(c_hbm, cloc)

    def _zrow(i, _):
        for k in range(8):
            rowbuf[i, pl.ds(k * 16, 16)] = zerov
        return ()
    lax.fori_loop(0, 128, _zrow, ())

    def _zden(i, _):
        zbuf[pl.ds(i * 16, 16)] = zerov
        return ()
    lax.fori_loop(0, 2000 // 16, _zden, ())

    rbase = sid * 632
    nchunk = jnp.where(sid < 15, 79, 65)

    def _zacc(i, _):
        pltpu.sync_copy(rowbuf.at[pl.ds(0, 8)],
                        acc_sh.at[pl.ds(rbase + i * 8, 8)])
        return ()
    lax.fori_loop(0, nchunk, _zacc, ())

    @pl.when(sid == 0)
    def _():
        for t in range(5):
            pltpu.sync_copy(zbuf, denom_sh.at[pl.ds(t * 2000, 2000)])

    C = cloc[pl.ds(0, 16)][0]

    ebase = sid * EPT

    def _pbody(j, _):
        cps = pltpu.async_copy(s_hbm.at[src2d.at[j]], svals, sem)
        cpd = pltpu.async_copy(d_hbm.at[dst2d.at[j]], dvals, sem)
        cps.wait()
        cpd.wait()
        for k in range(8):
            sv = svals[pl.ds(k * 16, 16)]
            dv = dvals[pl.ds(k * 16, 16)]
            e = sv + dv
            e = jnp.where(e >= 0.0, e, 0.2 * e)
            p = jnp.exp(e - C)
            gid = ebase + j * 128 + k * 16 + lax.iota(jnp.int32, 16)
            p2d[j, pl.ds(k * 16, 16)] = jnp.where(gid < E, p, 0.0)
        return ()
    lax.fori_loop(0, NB, _pbody, ())

    plsc.subcore_barrier()

    def _dscat(j, _):
        pltpu.sync_copy(p2d.at[j], denom_sh.at[dst2d.at[j]], add=True)
        return ()
    lax.fori_loop(0, NB, _dscat, ())

    plsc.subcore_barrier()

    off = c * N

    def _abody(j, _):
        pltpu.async_copy(denom_sh.at[dst2d.at[j]], dvals, sem).wait()
        for k in range(8):
            dv = dvals[pl.ds(k * 16, 16)]
            pv = p2d[j, pl.ds(k * 16, 16)]
            p2d[j, pl.ds(k * 16, 16)] = pv / (dv + 1e-16)
            src2d[j, pl.ds(k * 16, 16)] = src2d[j, pl.ds(k * 16, 16)] + off
        return ()
    lax.fori_loop(0, NB, _abody, ())

    def _gbody(j, _):
        pltpu.async_copy(h_hbm.at[src2d.at[j]], rowbuf, sem).wait()

        def _mul(g, _):
            av = p2d[j, pl.ds(g * 16, 16)]
            base = g * 16
            for ii in range(16):
                a = av[ii]
                r = base + ii
                for k in range(8):
                    rowbuf[r, pl.ds(k * 16, 16)] = (
                        rowbuf[r, pl.ds(k * 16, 16)] * a)
            return ()
        lax.fori_loop(0, 8, _mul, ())
        pltpu.sync_copy(rowbuf, acc_sh.at[dst2d.at[j]], add=True)
        return ()
    lax.fori_loop(0, NB, _gbody, ())

    plsc.subcore_barrier()

    @pl.when(sid < 15)
    def _():
        pltpu.sync_copy(acc_sh.at[pl.ds(rbase, 632)],
                        out_hbm.at[pl.ds(c * N + rbase, 632)])

    @pl.when(sid == 15)
    def _():
        pltpu.sync_copy(acc_sh.at[pl.ds(15 * 632, 520)],
                        out_hbm.at[pl.ds(c * N + 15 * 632, 520)])


_sc_edge = functools.partial(
    pl.kernel,
    mesh=plsc.VectorSubcoreMesh(core_axis_name="c", subcore_axis_name="s"),
    compiler_params=pltpu.CompilerParams(needs_layout_passes=False),
    out_type=jax.ShapeDtypeStruct((2 * N, FH), jnp.float32),
    scratch_types=[
        pltpu.VMEM((NB, 128), jnp.int32),
        pltpu.VMEM((NB, 128), jnp.int32),
        pltpu.VMEM((NB, 128), jnp.float32),
        pltpu.VMEM((128,), jnp.float32),
        pltpu.VMEM((128,), jnp.float32),
        pltpu.VMEM((2000,), jnp.float32),
        pltpu.VMEM((128, FH), jnp.float32),
        pltpu.VMEM((16,), jnp.float32),
        pltpu.VMEM_SHARED((N,), jnp.float32),
        pltpu.VMEM_SHARED((N, FH), jnp.float32),
        pltpu.SemaphoreType.DMA,
    ],
)(_sc_edge_body)


def _sc_layer(h_split, sd, cm, srcp, dstp):
    h_flat = h_split.reshape(2 * N, FH)
    cvec = cm.reshape(-1)[:16]
    agg = _sc_edge(h_flat, sd[:, 0], sd[:, 1], cvec, srcp, dstp)
    return agg.reshape(2, N, FH)


def kernel(x, edge_index, W0, a_src0, a_dst0, b0, W1, a_src1, a_dst1, b1,
           W2, a_src2, a_dst2, b2):
    pad = jnp.zeros((E_PAD - E,), jnp.int32)
    srcp = jnp.concatenate([edge_index[0], pad]).reshape(NT, NB, 128)
    dstp = jnp.concatenate([edge_index[1], pad]).reshape(NT, NB, 128)

    h0, sd0, cm0 = _tc_proj(x, W0, a_src0, a_dst0)
    agg0 = _sc_layer(h0, sd0, cm0, srcp, dstp)
    y1, h1, sd1, cm1 = _tc_epi_proj(None, agg0, b0, W1, a_src1, a_dst1)
    agg1 = _sc_layer(h1, sd1, cm1, srcp, dstp)
    y2, h2, sd2, cm2 = _tc_epi_proj(y1, agg1, b1, W2, a_src2, a_dst2)
    agg2 = _sc_layer(h2, sd2, cm2, srcp, dstp)
    return _tc_final(y2, agg2, b2)
